# Initial kernel scaffold; baseline (speedup 1.0000x reference)
import jax
import jax.numpy as jnp
from jax.experimental import pallas as pl


def kernel(sequence_output, sbo_positions, fc_w, fc_b):
    raise NotImplementedError("write your pallas kernel here")


# speedup vs baseline: 1.0019x; 1.0019x over previous
import functools

import jax
import jax.numpy as jnp
from jax import lax
from jax.experimental import pallas as pl
from jax.experimental.pallas import tpu as pltpu
from jax.experimental.pallas import tpu_sc as plsc

_B, _S, _H = 4, 4096, 1024
_P = 512
_NPOS = _B * _P
_WIN = _B * _S
_LANES = 16


def _make_sc_gather():
    info = plsc.get_sparse_core_info()
    nc, ns = info.num_cores, info.num_subcores
    nw = nc * ns
    per_w = _NPOS // nw
    half = per_w // 2
    mesh = plsc.VectorSubcoreMesh(core_axis_name="c", subcore_axis_name="s")

    @functools.partial(
        pl.kernel,
        mesh=mesh,
        out_type=(
            jax.ShapeDtypeStruct((_NPOS // 2,), jnp.float32),
            jax.ShapeDtypeStruct((_NPOS // 2,), jnp.float32),
        ),
        compiler_params=pltpu.CompilerParams(needs_layout_passes=False),
        scratch_types=[
            pltpu.VMEM((_WIN,), jnp.float32),
            pltpu.VMEM((per_w,), jnp.int32),
            pltpu.VMEM((half,), jnp.float32),
            pltpu.VMEM((half,), jnp.float32),
        ],
    )
    def sc_gather(seq_hbm, sbo_hbm, left_hbm, right_hbm,
                  win_v, sbo_v, left_v, right_v):
        wid = lax.axis_index("s") * nc + lax.axis_index("c")
        base = wid * per_w
        batch = base // _P
        off = batch * _S
        pltpu.sync_copy(seq_hbm.at[pl.ds(0, _WIN)], win_v)
        pltpu.sync_copy(sbo_hbm.at[pl.ds(base, per_w)], sbo_v)
        lanes = lax.iota(jnp.int32, _LANES)
        for k in range(half // _LANES):
            eidx = 2 * _LANES * k + 2 * lanes
            pos = plsc.load_gather(sbo_v, [eidx]) + off
            left_v[pl.ds(_LANES * k, _LANES)] = plsc.load_gather(win_v, [pos])
            pos = plsc.load_gather(sbo_v, [eidx + 1]) + off
            right_v[pl.ds(_LANES * k, _LANES)] = plsc.load_gather(win_v, [pos])
        pltpu.sync_copy(left_v, left_hbm.at[pl.ds(wid * half, half)])
        pltpu.sync_copy(right_v, right_hbm.at[pl.ds(wid * half, half)])

    return sc_gather


_SC_GATHER = _make_sc_gather()


def kernel(sequence_output, sbo_positions, fc_w, fc_b):
    seq_flat = sequence_output.reshape(-1)
    sbo_flat = sbo_positions.reshape(-1)
    left, right = _SC_GATHER(seq_flat, sbo_flat)
    return left.reshape(-1, 1), right.reshape(-1, 1)
